# baseline (speedup 1.0000x reference)
import functools

import jax
import jax.numpy as jnp
from jax import lax
from jax.experimental import pallas as pl
from jax.experimental.pallas import tpu as pltpu
from jax.experimental.pallas import tpu_sc as plsc

RB = 4
NBUF = 4
CH = 128


def _make_gather_kernel(B, H, D, mesh):
    nc = mesh.num_cores
    nw = nc * mesh.num_subcores
    rows_w = B // nw
    nch = rows_w // RB

    @functools.partial(
        pl.kernel,
        out_type=jax.ShapeDtypeStruct((B * H, 128), jnp.float32),
        mesh=mesh,
        compiler_params=pltpu.CompilerParams(
            needs_layout_passes=False, use_tc_tiling_on_sc=False,
            disable_bounds_checks=True, disable_semaphore_checks=True),
        scratch_types=[
            pltpu.VMEM((NBUF, RB, H), jnp.int32),
            pltpu.VMEM((NBUF, RB * H, 128), jnp.float32),
            [pltpu.SemaphoreType.DMA] * NBUF,
            [pltpu.SemaphoreType.DMA] * NBUF,
        ],
    )
    def k(ids_hbm, table_hbm, out_hbm, idx_v, rows_v, sem_g, sem_w):
        wid = lax.axis_index("s") * nc + lax.axis_index("c")
        brow0 = wid * rows_w

        def g_copies(s):
            return [pltpu.make_async_copy(
                table_hbm.at[idx_v.at[s, r]],
                rows_v.at[s, pl.ds(r * H, H)], sem_g[s])
                for r in range(RB)]

        def issue(s, c):
            br = brow0 + c * RB
            pltpu.sync_copy(ids_hbm.at[pl.ds(br, RB)], idx_v.at[s])
            for cp in g_copies(s):
                cp.start()

        def wait_g(s):
            for cp in g_copies(s):
                cp.wait()

        def w_copy(s, c):
            wr = (brow0 + c * RB) * H
            return pltpu.make_async_copy(
                rows_v.at[s], out_hbm.at[pl.ds(wr, RB * H)], sem_w[s])

        for s in range(NBUF - 1):
            issue(s, s)

        def blk(t, _):
            for b in range(NBUF):
                c = t * NBUF + b
                wait_g(b)
                w_copy(b, c).start()
                s = (b + NBUF - 1) % NBUF

                @pl.when(c >= 1)
                def _():
                    w_copy(s, c - 1).wait()

                @pl.when(c + NBUF - 1 < nch)
                def _():
                    issue(s, c + NBUF - 1)
            return 0

        lax.fori_loop(0, nch // NBUF, blk, 0)
        w_copy((nch - 1) % NBUF, nch - 1).wait()

    return k


def _make_ln_kernel(H, D):
    def _ln_kernel(x_ref, g_ref, b_ref, o_ref):
        x = x_ref[...].reshape(-1, H, 128)[:, :, :D]
        mean = jnp.mean(x, axis=-1, keepdims=True)
        var = jnp.mean(jnp.square(x), axis=-1, keepdims=True) - jnp.square(mean)
        inv = lax.rsqrt(var + 1e-5)
        g = g_ref[...].reshape(1, 1, -1)
        b = b_ref[...].reshape(1, 1, -1)
        o_ref[...] = (x - mean) * inv * g + b
    return _ln_kernel


def kernel(input_ids, table, gamma, beta):
    B, H = input_ids.shape
    V, D = table.shape
    if input_ids.dtype != jnp.int32:
        input_ids = input_ids.astype(jnp.int32)

    table128 = jnp.pad(table, ((0, 0), (0, 128 - D)))

    mesh = plsc.VectorSubcoreMesh(core_axis_name="c", subcore_axis_name="s")
    gathered = _make_gather_kernel(B, H, D, mesh)(input_ids, table128)

    grid = (B // CH,)
    out = pl.pallas_call(
        _make_ln_kernel(H, D),
        grid=grid,
        in_specs=[
            pl.BlockSpec((CH * H, 128), lambda i: (i, 0)),
            pl.BlockSpec((1, D), lambda i: (0, 0)),
            pl.BlockSpec((1, D), lambda i: (0, 0)),
        ],
        out_specs=pl.BlockSpec((CH, H, D), lambda i: (i, 0, 0)),
        out_shape=jax.ShapeDtypeStruct((B, H, D), jnp.float32),
        compiler_params=pltpu.CompilerParams(
            dimension_semantics=("arbitrary",)),
    )(gathered, gamma.reshape(1, D), beta.reshape(1, D))
    return out


# speedup vs baseline: 1.0974x; 1.0974x over previous
import functools

import jax
import jax.numpy as jnp
from jax import lax
from jax.experimental import pallas as pl
from jax.experimental.pallas import tpu as pltpu
from jax.experimental.pallas import tpu_sc as plsc

RB = 8
NBUF = 4
CH = 128


def _make_gather_kernel(B, H, D, mesh):
    nc = mesh.num_cores
    nw = nc * mesh.num_subcores
    rows_w = B // nw
    nch = rows_w // RB

    @functools.partial(
        pl.kernel,
        out_type=jax.ShapeDtypeStruct((B, H, D), jnp.float32),
        mesh=mesh,
        compiler_params=pltpu.CompilerParams(
            needs_layout_passes=False, use_tc_tiling_on_sc=False,
            disable_bounds_checks=True, disable_semaphore_checks=True),
        scratch_types=[
            pltpu.VMEM((NBUF, RB, H), jnp.int32),
            pltpu.VMEM((NBUF, RB, H, D), jnp.float32),
            [pltpu.SemaphoreType.DMA] * NBUF,
            [pltpu.SemaphoreType.DMA] * NBUF,
        ],
    )
    def k(ids_hbm, table_hbm, out_hbm, idx_v, rows_v, sem_g, sem_w):
        wid = lax.axis_index("s") * nc + lax.axis_index("c")
        brow0 = wid * rows_w

        def g_copies(s):
            return [pltpu.make_async_copy(
                table_hbm.at[idx_v.at[s, r]], rows_v.at[s, r], sem_g[s])
                for r in range(RB)]

        def issue(s, c):
            br = brow0 + c * RB
            pltpu.sync_copy(ids_hbm.at[pl.ds(br, RB)], idx_v.at[s])
            for cp in g_copies(s):
                cp.start()

        def wait_g(s):
            for cp in g_copies(s):
                cp.wait()

        def w_copy(s, c):
            br = brow0 + c * RB
            return pltpu.make_async_copy(
                rows_v.at[s], out_hbm.at[pl.ds(br, RB)], sem_w[s])

        for s in range(NBUF - 1):
            issue(s, s)

        def blk(t, _):
            for b in range(NBUF):
                c = t * NBUF + b
                wait_g(b)
                w_copy(b, c).start()
                s = (b + NBUF - 1) % NBUF

                @pl.when(c >= 1)
                def _():
                    w_copy(s, c - 1).wait()

                @pl.when(c + NBUF - 1 < nch)
                def _():
                    issue(s, c + NBUF - 1)
            return 0

        lax.fori_loop(0, nch // NBUF, blk, 0)
        w_copy((nch - 1) % NBUF, nch - 1).wait()

    return k


def _ln_kernel(x_ref, g_ref, b_ref, o_ref):
    x = x_ref[...]
    mean = jnp.mean(x, axis=-1, keepdims=True)
    var = jnp.mean(jnp.square(x), axis=-1, keepdims=True) - jnp.square(mean)
    inv = lax.rsqrt(var + 1e-5)
    g = g_ref[...].reshape(1, 1, -1)
    b = b_ref[...].reshape(1, 1, -1)
    o_ref[...] = (x - mean) * inv * g + b


def kernel(input_ids, table, gamma, beta):
    B, H = input_ids.shape
    V, D = table.shape
    if input_ids.dtype != jnp.int32:
        input_ids = input_ids.astype(jnp.int32)

    mesh = plsc.VectorSubcoreMesh(core_axis_name="c", subcore_axis_name="s")
    gathered = _make_gather_kernel(B, H, D, mesh)(input_ids, table)

    grid = (B // CH,)
    out = pl.pallas_call(
        _ln_kernel,
        grid=grid,
        in_specs=[
            pl.BlockSpec((CH, H, D), lambda i: (i, 0, 0)),
            pl.BlockSpec((1, D), lambda i: (0, 0)),
            pl.BlockSpec((1, D), lambda i: (0, 0)),
        ],
        out_specs=pl.BlockSpec((CH, H, D), lambda i: (i, 0, 0)),
        out_shape=jax.ShapeDtypeStruct((B, H, D), jnp.float32),
        input_output_aliases={0: 0},
        compiler_params=pltpu.CompilerParams(
            dimension_semantics=("arbitrary",)),
    )(gathered, gamma.reshape(1, D), beta.reshape(1, D))
    return out
